# Initial kernel scaffold; baseline (speedup 1.0000x reference)
import jax
import jax.numpy as jnp
from jax.experimental import pallas as pl


def kernel(uids, iids, i_user_pad, i_friends_pad, i_friends_user_pad, params):
    raise NotImplementedError("write your pallas kernel here")


# speedup vs baseline: 3.9811x; 3.9811x over previous
import functools

import jax
import jax.numpy as jnp
from jax import lax
from jax.experimental import pallas as pl
from jax.experimental.pallas import tpu as pltpu
from jax.experimental.pallas import tpu_sc as plsc

D = 64
_NC = 2
_NS = 16
_NW = _NC * _NS
_EPS = 1e-10
_BLK = 32


def _sc_gather_all(ue, ie, re_, i_pt, i_er, i_qa, i_qas, i_ers, i_iid):
    n0, n1, n2, n3 = i_pt.shape[0], i_qa.shape[0], i_qas.shape[0], i_iid.shape[0]
    mesh = plsc.VectorSubcoreMesh(core_axis_name="c", subcore_axis_name="s")
    out_type = [
        jax.ShapeDtypeStruct((n0, 2 * D), jnp.float32),
        jax.ShapeDtypeStruct((n1, D), jnp.float32),
        jax.ShapeDtypeStruct((n2, 2 * D), jnp.float32),
        jax.ShapeDtypeStruct((n3, D), jnp.float32),
    ]
    C0, C1, C2 = 80, 80, 128
    scratch = [
        pltpu.VMEM((2, C0), jnp.int32), pltpu.VMEM((2, C0), jnp.int32),
        pltpu.VMEM((2, C0, D), jnp.float32), pltpu.VMEM((2, C0, D), jnp.float32),
        pltpu.VMEM((2, C1), jnp.int32), pltpu.VMEM((2, C1, D), jnp.float32),
        pltpu.VMEM((2, C2), jnp.int32), pltpu.VMEM((2, C2), jnp.int32),
        pltpu.VMEM((2, C2, D), jnp.float32), pltpu.VMEM((2, C2, D), jnp.float32),
        pltpu.VMEM((n3 // _NW,), jnp.int32), pltpu.VMEM((n3 // _NW, D), jnp.float32),
        pltpu.SemaphoreType.DMA, pltpu.SemaphoreType.DMA, pltpu.SemaphoreType.DMA,
    ]

    @functools.partial(
        pl.kernel,
        mesh=mesh,
        out_type=out_type,
        scratch_types=scratch,
        compiler_params=pltpu.CompilerParams(use_tc_tiling_on_sc=False),
    )
    def k(ue_h, ie_h, re_h, ipt_h, ier_h, iqa_h, iqas_h, iers_h, iiid_h,
          o0, o1, o2, o3,
          ia0, ib0, ra0, rb0, ia1, r1, ia2, ib2, ra2, rb2, ia3, r3,
          sem0, sem1, semiid):
        wid = lax.axis_index("s") * _NC + lax.axis_index("c")
        sems = (sem0, sem1)

        def pair_loop(C, tabA, tabB, idxA_h, idxB_h, rA, rB, iA, iB, out_h, pw):
            base = wid * pw
            nch = pw // C

            def start(ci, b):
                off = base + ci * C
                pltpu.sync_copy(idxA_h.at[pl.ds(off, C)], iA.at[b])
                pltpu.sync_copy(idxB_h.at[pl.ds(off, C)], iB.at[b])
                cpa = pltpu.async_copy(tabA.at[iA.at[b]], rA.at[b], sems[b])
                cpb = pltpu.async_copy(tabB.at[iB.at[b]], rB.at[b], sems[b])
                return cpa, cpb

            def fin(ci, b, cps):
                cps[0].wait()
                cps[1].wait()
                off = base + ci * C
                pltpu.sync_copy(rA.at[b], out_h.at[pl.ds(off, C), pl.ds(0, D)])
                pltpu.sync_copy(rB.at[b], out_h.at[pl.ds(off, C), pl.ds(D, D)])

            def body(j, carry):
                c0 = 2 * j
                cp0 = start(c0, 0)
                cp1 = start(c0 + 1, 1)
                fin(c0, 0, cp0)
                fin(c0 + 1, 1, cp1)
                return carry

            lax.fori_loop(0, nch // 2, body, 0)

        def single_loop(C, tab, idx_h, rows, iA, out_h, pw):
            base = wid * pw
            nch = pw // C

            def start(ci, b):
                off = base + ci * C
                pltpu.sync_copy(idx_h.at[pl.ds(off, C)], iA.at[b])
                return pltpu.async_copy(tab.at[iA.at[b]], rows.at[b], sems[b])

            def body(j, carry):
                c0 = 2 * j
                cp0 = start(c0, 0)
                cp1 = start(c0 + 1, 1)
                cp0.wait()
                pltpu.sync_copy(rows.at[0], out_h.at[pl.ds(base + c0 * C, C)])
                cp1.wait()
                pltpu.sync_copy(rows.at[1], out_h.at[pl.ds(base + (c0 + 1) * C, C)])
                return carry

            lax.fori_loop(0, nch // 2, body, 0)

        pair_loop(C0, ue_h, re_h, ipt_h, ier_h, ra0, rb0, ia0, ib0, o0, n0 // _NW)
        single_loop(C1, ie_h, iqa_h, r1, ia1, o1, n1 // _NW)
        pair_loop(C2, ue_h, re_h, iqas_h, iers_h, ra2, rb2, ia2, ib2, o2, n2 // _NW)
        pwi = n3 // _NW
        basei = wid * pwi
        pltpu.sync_copy(iiid_h.at[pl.ds(basei, pwi)], ia3)
        pltpu.async_copy(ie_h.at[ia3], r3, semiid).wait()
        pltpu.sync_copy(r3, o3.at[pl.ds(basei, pwi)])

    return k(ue, ie, re_, i_pt, i_er, i_qa, i_qas, i_ers, i_iid)


def _leaky(h):
    return jnp.where(h > 0, h, 0.2 * h)


def _dot(a, b):
    return jnp.dot(a, b, preferred_element_type=jnp.float32)


def _tc_body(U, F, S,
             x1_r, xq_r, x3_r, iid_r, m1_r, mu_r, ms_r,
             gu1_r, gub_r, gu2_r, gu2b_r,
             w1_r, w1b_r, a1A_r, a1B_r, a1b_r, a1v_r, agUi_r, agUib_r,
             w2_r, w2b_r, f1A_r, f1B_r, f1b_r, f1v_r, agIF_r, agIFb_r,
             w3_r, w3b_r, u1A_r, u1B_r, u1b_r, u1v_r, agU_r, agUb_r,
             w4_r, w4b_r, i1A_r, i1B_r, i1b_r, i1v_r, agFF_r, agFFb_r,
             c1A_r, c1B_r, c1C_r, c1b_r, c2_r, c2b_r, a2b_r,
             o_ref):
    blk = iid_r.shape[1]
    iidv = iid_r[0]
    a2b = a2b_r[...]
    gu1, gub, gu2, gu2b = gu1_r[...], gub_r[...], gu2_r[...], gu2b_r[...]

    x1 = jnp.reshape(x1_r[...], (U * blk, 2 * D))
    h = _leaky(_dot(x1, gu1) + gub)
    fjt = _dot(h, gu2) + gu2b
    w1f = _dot(fjt, w1_r[...]) + w1b_r[...]
    u1 = _dot(w1f, a1A_r[...])
    u2 = _dot(_dot(iidv, w1_r[...]), a1B_r[...])
    cc = _dot(w1b_r[...], a1B_r[...]) + a1b_r[...]
    ha = _leaky(jnp.reshape(u1, (U, blk, D)) + u2[None, :, :] + cc[None, :, :])
    lg = _dot(jnp.reshape(ha, (U * blk, D)), a1v_r[...]) + a2b[0, 0]
    e = jnp.exp(jnp.reshape(lg, (U, blk, 1))) * m1_r[...]
    w = e / (jnp.sum(e, axis=0, keepdims=True) + _EPS)
    zj_pre = jnp.sum(w * jnp.reshape(w1f, (U, blk, D)), axis=0)
    z_j = jax.nn.relu(_dot(zj_pre, agUi_r[...]) + agUib_r[...])

    xq = jnp.reshape(xq_r[...], (F * blk, D))
    w2q = _dot(xq, w2_r[...]) + w2b_r[...]
    u1 = _dot(w2q, f1A_r[...])
    u2 = _dot(_dot(iidv, w2_r[...]), f1B_r[...])
    cc = _dot(w2b_r[...], f1B_r[...]) + f1b_r[...]
    ha = _leaky(jnp.reshape(u1, (F, blk, D)) + u2[None, :, :] + cc[None, :, :])
    lg = _dot(jnp.reshape(ha, (F * blk, D)), f1v_r[...]) + a2b[0, 1]
    mu = mu_r[...]
    e = jnp.exp(jnp.reshape(lg, (F, blk, 1))) * mu
    w = e / (jnp.sum(e, axis=0, keepdims=True) + _EPS)
    zif_pre = jnp.sum(w * jnp.reshape(w2q, (F, blk, D)), axis=0)
    z_if = jax.nn.relu(_dot(zif_pre, agIF_r[...]) + agIFb_r[...])

    x3 = jnp.reshape(x3_r[...], (S * F * blk, 2 * D))
    h = _leaky(_dot(x3, gu1) + gub)
    xias = _dot(h, gu2) + gu2b
    w3x = _dot(xias, w3_r[...]) + w3b_r[...]
    u1 = _dot(w3x, u1A_r[...])
    u2 = _dot(_dot(xq, w3_r[...]), u1B_r[...])
    cc = _dot(w3b_r[...], u1B_r[...]) + u1b_r[...]
    ha = _leaky(jnp.reshape(u1, (S, F * blk, D)) + u2[None, :, :] + cc[None, :, :])
    lg = _dot(jnp.reshape(ha, (S * F * blk, D)), u1v_r[...]) + a2b[0, 2]
    ms = ms_r[...]
    e = jnp.exp(jnp.reshape(lg, (S, F, blk, 1))) * jnp.reshape(ms, (S, F, blk, 1))
    w = e / (jnp.sum(e, axis=0, keepdims=True) + _EPS)
    hoI_pre = jnp.sum(w * jnp.reshape(w3x, (S, F, blk, D)), axis=0)
    h_oI = jax.nn.relu(_dot(jnp.reshape(hoI_pre, (F * blk, D)), agU_r[...]) + agUb_r[...])

    w4h = _dot(h_oI, w4_r[...]) + w4b_r[...]
    u1 = _dot(w4h, i1A_r[...])
    u2 = _dot(_dot(iidv, w4_r[...]), i1B_r[...])
    cc = _dot(w4b_r[...], i1B_r[...]) + i1b_r[...]
    ha = _leaky(jnp.reshape(u1, (F, blk, D)) + u2[None, :, :] + cc[None, :, :])
    lg = _dot(jnp.reshape(ha, (F * blk, D)), i1v_r[...]) + a2b[0, 3]
    e = jnp.exp(jnp.reshape(lg, (F, blk, 1))) * mu
    w = e / (jnp.sum(e, axis=0, keepdims=True) + _EPS)
    zuf_pre = jnp.sum(w * jnp.reshape(h_oI, (F, blk, D)), axis=0)
    z_uf = jax.nn.relu(_dot(zuf_pre, agFF_r[...]) + agFFb_r[...])

    zc = jax.nn.relu(_dot(z_if, c1A_r[...]) + _dot(z_j, c1B_r[...])
                     + _dot(z_uf, c1C_r[...]) + c1b_r[...])
    o_ref[...] = jax.nn.relu(_dot(zc, c2_r[...]) + c2b_r[...])


def _wt(p):
    return p["W"].T


def _bt(p):
    return p["b"].reshape(1, -1)


def kernel(uids, iids, i_user_pad, i_friends_pad, i_friends_user_pad, params):
    B, U, _ = i_user_pad.shape
    F = i_friends_pad.shape[1]
    S = i_friends_user_pad.shape[2]

    i_pt = i_user_pad[:, :, 0].T.reshape(-1).astype(jnp.int32)
    i_er = i_user_pad[:, :, 1].T.reshape(-1).astype(jnp.int32)
    i_qa = i_friends_pad[:, :, 0].T.reshape(-1).astype(jnp.int32)
    i_qas = jnp.transpose(i_friends_user_pad[:, :, :, 0], (2, 1, 0)).reshape(-1).astype(jnp.int32)
    i_ers = jnp.transpose(i_friends_user_pad[:, :, :, 1], (2, 1, 0)).reshape(-1).astype(jnp.int32)
    i_iid = iids.reshape(-1).astype(jnp.int32)

    x1, xq, x3, g_iid = _sc_gather_all(
        params["user_emb"], params["item_emb"], params["rate_emb"],
        i_pt, i_er, i_qa, i_qas, i_ers, i_iid)

    x1 = x1.reshape(U, B, 2 * D)
    xq = xq.reshape(F, B, D)
    x3 = x3.reshape(S * F, B, 2 * D)
    g_iid = g_iid.reshape(1, B, D)

    m1 = (i_user_pad[:, :, 0] > 0).T.astype(jnp.float32).reshape(U, B, 1)
    mu = (i_friends_pad[:, :, 0] > 0).T.astype(jnp.float32).reshape(F, B, 1)
    ms = (jnp.transpose(i_friends_user_pad[:, :, :, 0], (2, 1, 0)) > 0) \
        .astype(jnp.float32).reshape(S * F, B, 1)

    p = params
    a2b = jnp.concatenate([
        p["item_users_att_i"]["l2"]["b"], p["i_friends_att"]["l2"]["b"],
        p["item_users_att"]["l2"]["b"], p["if_friends_att"]["l2"]["b"]]).reshape(1, 4)
    a1t = _wt(p["item_users_att_i"]["l1"])
    f1t = _wt(p["i_friends_att"]["l1"])
    u1t = _wt(p["item_users_att"]["l1"])
    i1t = _wt(p["if_friends_att"]["l1"])
    c1t = _wt(p["combine_l1"])

    weights = [
        _wt(p["g_u"]["l1"]), _bt(p["g_u"]["l1"]), _wt(p["g_u"]["l2"]), _bt(p["g_u"]["l2"]),
        _wt(p["w1"]), _bt(p["w1"]),
        a1t[:D], a1t[D:], _bt(p["item_users_att_i"]["l1"]),
        _wt(p["item_users_att_i"]["l2"]),
        _wt(p["aggre_users_i"]), _bt(p["aggre_users_i"]),
        _wt(p["w2"]), _bt(p["w2"]),
        f1t[:D], f1t[D:], _bt(p["i_friends_att"]["l1"]),
        _wt(p["i_friends_att"]["l2"]),
        _wt(p["aggre_i_friends"]), _bt(p["aggre_i_friends"]),
        _wt(p["w3"]), _bt(p["w3"]),
        u1t[:D], u1t[D:], _bt(p["item_users_att"]["l1"]),
        _wt(p["item_users_att"]["l2"]),
        _wt(p["aggre_users"]), _bt(p["aggre_users"]),
        _wt(p["w4"]), _bt(p["w4"]),
        i1t[:D], i1t[D:], _bt(p["if_friends_att"]["l1"]),
        _wt(p["if_friends_att"]["l2"]),
        _wt(p["aggre_if_friends"]), _bt(p["aggre_if_friends"]),
        c1t[:D], c1t[D:2 * D], c1t[2 * D:], _bt(p["combine_l1"]),
        _wt(p["combine_l2"]), _bt(p["combine_l2"]), a2b,
    ]

    blk = _BLK
    grid = B // blk

    def bspec(lead, width):
        return pl.BlockSpec((lead, blk, width), lambda i: (0, i, 0))

    def mspec(lead):
        return pl.BlockSpec((lead, blk, 1), lambda i: (0, i, 0))

    def full_spec(a):
        return pl.BlockSpec(a.shape, lambda i: tuple(0 for _ in a.shape))

    in_specs = [
        bspec(U, 2 * D), bspec(F, D), bspec(S * F, 2 * D), bspec(1, D),
        mspec(U), mspec(F), mspec(S * F),
    ] + [full_spec(wa) for wa in weights]

    out = pl.pallas_call(
        functools.partial(_tc_body, U, F, S),
        grid=(grid,),
        in_specs=in_specs,
        out_specs=pl.BlockSpec((blk, D), lambda i: (i, 0)),
        out_shape=jax.ShapeDtypeStruct((B, D), jnp.float32),
    )(x1, xq, x3, g_iid, m1, mu, ms, *weights)
    return out


# speedup vs baseline: 4.2041x; 1.0560x over previous
import functools

import jax
import jax.numpy as jnp
from jax import lax
from jax.experimental import pallas as pl
from jax.experimental.pallas import tpu as pltpu
from jax.experimental.pallas import tpu_sc as plsc

D = 64
_NC = 2
_NS = 16
_NW = _NC * _NS
_EPS = 1e-10
_BLK = 64


def _sc_gather_all(ue, ie, re_, i_pt, i_er, i_qa, i_qas, i_ers, i_iid):
    n0, n1, n2, n3 = i_pt.shape[0], i_qa.shape[0], i_qas.shape[0], i_iid.shape[0]
    mesh = plsc.VectorSubcoreMesh(core_axis_name="c", subcore_axis_name="s")
    out_type = [
        jax.ShapeDtypeStruct((n0, 2 * D), jnp.float32),
        jax.ShapeDtypeStruct((n1, D), jnp.float32),
        jax.ShapeDtypeStruct((n2, 2 * D), jnp.float32),
        jax.ShapeDtypeStruct((n3, D), jnp.float32),
    ]
    C0, C1, C2 = 80, 80, 128
    scratch = [
        pltpu.VMEM((2, C0), jnp.int32), pltpu.VMEM((2, C0), jnp.int32),
        pltpu.VMEM((2, C0, D), jnp.float32), pltpu.VMEM((2, C0, D), jnp.float32),
        pltpu.VMEM((2, C1), jnp.int32), pltpu.VMEM((2, C1, D), jnp.float32),
        pltpu.VMEM((2, C2), jnp.int32), pltpu.VMEM((2, C2), jnp.int32),
        pltpu.VMEM((2, C2, D), jnp.float32), pltpu.VMEM((2, C2, D), jnp.float32),
        pltpu.VMEM((n3 // _NW,), jnp.int32), pltpu.VMEM((n3 // _NW, D), jnp.float32),
        pltpu.SemaphoreType.DMA, pltpu.SemaphoreType.DMA, pltpu.SemaphoreType.DMA,
    ]

    @functools.partial(
        pl.kernel,
        mesh=mesh,
        out_type=out_type,
        scratch_types=scratch,
        compiler_params=pltpu.CompilerParams(use_tc_tiling_on_sc=False),
    )
    def k(ue_h, ie_h, re_h, ipt_h, ier_h, iqa_h, iqas_h, iers_h, iiid_h,
          o0, o1, o2, o3,
          ia0, ib0, ra0, rb0, ia1, r1, ia2, ib2, ra2, rb2, ia3, r3,
          sem0, sem1, semiid):
        wid = lax.axis_index("s") * _NC + lax.axis_index("c")
        sems = (sem0, sem1)

        def pair_loop(C, tabA, tabB, idxA_h, idxB_h, rA, rB, iA, iB, out_h, pw):
            base = wid * pw
            nch = pw // C

            def start(ci, b):
                off = base + ci * C
                pltpu.sync_copy(idxA_h.at[pl.ds(off, C)], iA.at[b])
                pltpu.sync_copy(idxB_h.at[pl.ds(off, C)], iB.at[b])
                cpa = pltpu.async_copy(tabA.at[iA.at[b]], rA.at[b], sems[b])
                cpb = pltpu.async_copy(tabB.at[iB.at[b]], rB.at[b], sems[b])
                return cpa, cpb

            def fin(ci, b, cps):
                cps[0].wait()
                cps[1].wait()
                off = base + ci * C
                pltpu.sync_copy(rA.at[b], out_h.at[pl.ds(off, C), pl.ds(0, D)])
                pltpu.sync_copy(rB.at[b], out_h.at[pl.ds(off, C), pl.ds(D, D)])

            def body(j, carry):
                c0 = 2 * j
                cp0 = start(c0, 0)
                cp1 = start(c0 + 1, 1)
                fin(c0, 0, cp0)
                fin(c0 + 1, 1, cp1)
                return carry

            lax.fori_loop(0, nch // 2, body, 0)

        def single_loop(C, tab, idx_h, rows, iA, out_h, pw):
            base = wid * pw
            nch = pw // C

            def start(ci, b):
                off = base + ci * C
                pltpu.sync_copy(idx_h.at[pl.ds(off, C)], iA.at[b])
                return pltpu.async_copy(tab.at[iA.at[b]], rows.at[b], sems[b])

            def body(j, carry):
                c0 = 2 * j
                cp0 = start(c0, 0)
                cp1 = start(c0 + 1, 1)
                cp0.wait()
                pltpu.sync_copy(rows.at[0], out_h.at[pl.ds(base + c0 * C, C)])
                cp1.wait()
                pltpu.sync_copy(rows.at[1], out_h.at[pl.ds(base + (c0 + 1) * C, C)])
                return carry

            lax.fori_loop(0, nch // 2, body, 0)

        pair_loop(C0, ue_h, re_h, ipt_h, ier_h, ra0, rb0, ia0, ib0, o0, n0 // _NW)
        single_loop(C1, ie_h, iqa_h, r1, ia1, o1, n1 // _NW)
        pair_loop(C2, ue_h, re_h, iqas_h, iers_h, ra2, rb2, ia2, ib2, o2, n2 // _NW)
        pwi = n3 // _NW
        basei = wid * pwi
        pltpu.sync_copy(iiid_h.at[pl.ds(basei, pwi)], ia3)
        pltpu.async_copy(ie_h.at[ia3], r3, semiid).wait()
        pltpu.sync_copy(r3, o3.at[pl.ds(basei, pwi)])

    return k(ue, ie, re_, i_pt, i_er, i_qa, i_qas, i_ers, i_iid)


def _leaky(h):
    return jnp.maximum(h, 0.2 * h)


def _dot(a, b):
    return jnp.dot(a, b, preferred_element_type=jnp.float32)


def _tc_body(U, F, S,
             x1_r, xq_r, x3_r, iid_r, m1_r, mu_r, ms_r,
             gu1_r, gub_r, gu2_r, gu2b_r,
             w1_r, w1b_r, a1A_r, a1B_r, a1b_r, a1v_r, agUi_r, agUib_r,
             w2_r, w2b_r, f1A_r, f1B_r, f1b_r, f1v_r, agIF_r, agIFb_r,
             w3_r, w3b_r, u1A_r, u1B_r, u1b_r, u1v_r, agU_r, agUb_r,
             w4_r, w4b_r, i1A_r, i1B_r, i1b_r, i1v_r, agFF_r, agFFb_r,
             c1A_r, c1B_r, c1C_r, c1b_r, c2_r, c2b_r, a2b_r,
             o_ref):
    blk = iid_r.shape[1]
    iidv = iid_r[0]
    a2b = a2b_r[...]
    gu1, gub, gu2, gu2b = gu1_r[...], gub_r[...], gu2_r[...], gu2b_r[...]

    x1 = jnp.reshape(x1_r[...], (U * blk, 2 * D))
    h = _leaky(_dot(x1, gu1) + gub)
    fjt = _dot(h, gu2) + gu2b
    w1f = _dot(fjt, w1_r[...]) + w1b_r[...]
    u1 = _dot(w1f, a1A_r[...])
    u2 = _dot(_dot(iidv, w1_r[...]), a1B_r[...])
    cc = _dot(w1b_r[...], a1B_r[...]) + a1b_r[...]
    ha = _leaky(jnp.reshape(u1, (U, blk, D)) + u2[None, :, :] + cc[None, :, :])
    lg = _dot(jnp.reshape(ha, (U * blk, D)), a1v_r[...]) + a2b[0, 0]
    e = jnp.exp(jnp.reshape(lg, (U, blk, 1))) * m1_r[...]
    w = e / (jnp.sum(e, axis=0, keepdims=True) + _EPS)
    zj_pre = jnp.sum(w * jnp.reshape(w1f, (U, blk, D)), axis=0)
    z_j = jax.nn.relu(_dot(zj_pre, agUi_r[...]) + agUib_r[...])

    xq = jnp.reshape(xq_r[...], (F * blk, D))
    w2q = _dot(xq, w2_r[...]) + w2b_r[...]
    u1 = _dot(w2q, f1A_r[...])
    u2 = _dot(_dot(iidv, w2_r[...]), f1B_r[...])
    cc = _dot(w2b_r[...], f1B_r[...]) + f1b_r[...]
    ha = _leaky(jnp.reshape(u1, (F, blk, D)) + u2[None, :, :] + cc[None, :, :])
    lg = _dot(jnp.reshape(ha, (F * blk, D)), f1v_r[...]) + a2b[0, 1]
    mu = mu_r[...]
    e = jnp.exp(jnp.reshape(lg, (F, blk, 1))) * mu
    w = e / (jnp.sum(e, axis=0, keepdims=True) + _EPS)
    zif_pre = jnp.sum(w * jnp.reshape(w2q, (F, blk, D)), axis=0)
    z_if = jax.nn.relu(_dot(zif_pre, agIF_r[...]) + agIFb_r[...])

    x3 = jnp.reshape(x3_r[...], (S * F * blk, 2 * D))
    h = _leaky(_dot(x3, gu1) + gub)
    xias = _dot(h, gu2) + gu2b
    w3x = _dot(xias, w3_r[...]) + w3b_r[...]
    u1 = _dot(w3x, u1A_r[...])
    u2 = _dot(_dot(xq, w3_r[...]), u1B_r[...])
    cc = _dot(w3b_r[...], u1B_r[...]) + u1b_r[...]
    ha = _leaky(jnp.reshape(u1, (S, F * blk, D)) + u2[None, :, :] + cc[None, :, :])
    lg = _dot(jnp.reshape(ha, (S * F * blk, D)), u1v_r[...]) + a2b[0, 2]
    ms = ms_r[...]
    e = jnp.exp(jnp.reshape(lg, (S, F, blk, 1))) * jnp.reshape(ms, (S, F, blk, 1))
    w = e / (jnp.sum(e, axis=0, keepdims=True) + _EPS)
    hoI_pre = jnp.sum(w * jnp.reshape(w3x, (S, F, blk, D)), axis=0)
    h_oI = jax.nn.relu(_dot(jnp.reshape(hoI_pre, (F * blk, D)), agU_r[...]) + agUb_r[...])

    w4h = _dot(h_oI, w4_r[...]) + w4b_r[...]
    u1 = _dot(w4h, i1A_r[...])
    u2 = _dot(_dot(iidv, w4_r[...]), i1B_r[...])
    cc = _dot(w4b_r[...], i1B_r[...]) + i1b_r[...]
    ha = _leaky(jnp.reshape(u1, (F, blk, D)) + u2[None, :, :] + cc[None, :, :])
    lg = _dot(jnp.reshape(ha, (F * blk, D)), i1v_r[...]) + a2b[0, 3]
    e = jnp.exp(jnp.reshape(lg, (F, blk, 1))) * mu
    w = e / (jnp.sum(e, axis=0, keepdims=True) + _EPS)
    zuf_pre = jnp.sum(w * jnp.reshape(h_oI, (F, blk, D)), axis=0)
    z_uf = jax.nn.relu(_dot(zuf_pre, agFF_r[...]) + agFFb_r[...])

    zc = jax.nn.relu(_dot(z_if, c1A_r[...]) + _dot(z_j, c1B_r[...])
                     + _dot(z_uf, c1C_r[...]) + c1b_r[...])
    o_ref[...] = jax.nn.relu(_dot(zc, c2_r[...]) + c2b_r[...])


def _wt(p):
    return p["W"].T


def _bt(p):
    return p["b"].reshape(1, -1)


def kernel(uids, iids, i_user_pad, i_friends_pad, i_friends_user_pad, params):
    B, U, _ = i_user_pad.shape
    F = i_friends_pad.shape[1]
    S = i_friends_user_pad.shape[2]

    i_pt = i_user_pad[:, :, 0].T.reshape(-1).astype(jnp.int32)
    i_er = i_user_pad[:, :, 1].T.reshape(-1).astype(jnp.int32)
    i_qa = i_friends_pad[:, :, 0].T.reshape(-1).astype(jnp.int32)
    i_qas = jnp.transpose(i_friends_user_pad[:, :, :, 0], (2, 1, 0)).reshape(-1).astype(jnp.int32)
    i_ers = jnp.transpose(i_friends_user_pad[:, :, :, 1], (2, 1, 0)).reshape(-1).astype(jnp.int32)
    i_iid = iids.reshape(-1).astype(jnp.int32)

    x1, xq, x3, g_iid = _sc_gather_all(
        params["user_emb"], params["item_emb"], params["rate_emb"],
        i_pt, i_er, i_qa, i_qas, i_ers, i_iid)

    x1 = x1.reshape(U, B, 2 * D)
    xq = xq.reshape(F, B, D)
    x3 = x3.reshape(S * F, B, 2 * D)
    g_iid = g_iid.reshape(1, B, D)

    m1 = (i_user_pad[:, :, 0] > 0).T.astype(jnp.float32).reshape(U, B, 1)
    mu = (i_friends_pad[:, :, 0] > 0).T.astype(jnp.float32).reshape(F, B, 1)
    ms = (jnp.transpose(i_friends_user_pad[:, :, :, 0], (2, 1, 0)) > 0) \
        .astype(jnp.float32).reshape(S * F, B, 1)

    p = params
    a2b = jnp.concatenate([
        p["item_users_att_i"]["l2"]["b"], p["i_friends_att"]["l2"]["b"],
        p["item_users_att"]["l2"]["b"], p["if_friends_att"]["l2"]["b"]]).reshape(1, 4)
    a1t = _wt(p["item_users_att_i"]["l1"])
    f1t = _wt(p["i_friends_att"]["l1"])
    u1t = _wt(p["item_users_att"]["l1"])
    i1t = _wt(p["if_friends_att"]["l1"])
    c1t = _wt(p["combine_l1"])

    weights = [
        _wt(p["g_u"]["l1"]), _bt(p["g_u"]["l1"]), _wt(p["g_u"]["l2"]), _bt(p["g_u"]["l2"]),
        _wt(p["w1"]), _bt(p["w1"]),
        a1t[:D], a1t[D:], _bt(p["item_users_att_i"]["l1"]),
        _wt(p["item_users_att_i"]["l2"]),
        _wt(p["aggre_users_i"]), _bt(p["aggre_users_i"]),
        _wt(p["w2"]), _bt(p["w2"]),
        f1t[:D], f1t[D:], _bt(p["i_friends_att"]["l1"]),
        _wt(p["i_friends_att"]["l2"]),
        _wt(p["aggre_i_friends"]), _bt(p["aggre_i_friends"]),
        _wt(p["w3"]), _bt(p["w3"]),
        u1t[:D], u1t[D:], _bt(p["item_users_att"]["l1"]),
        _wt(p["item_users_att"]["l2"]),
        _wt(p["aggre_users"]), _bt(p["aggre_users"]),
        _wt(p["w4"]), _bt(p["w4"]),
        i1t[:D], i1t[D:], _bt(p["if_friends_att"]["l1"]),
        _wt(p["if_friends_att"]["l2"]),
        _wt(p["aggre_if_friends"]), _bt(p["aggre_if_friends"]),
        c1t[:D], c1t[D:2 * D], c1t[2 * D:], _bt(p["combine_l1"]),
        _wt(p["combine_l2"]), _bt(p["combine_l2"]), a2b,
    ]

    blk = _BLK
    grid = B // blk

    def bspec(lead, width):
        return pl.BlockSpec((lead, blk, width), lambda i: (0, i, 0))

    def mspec(lead):
        return pl.BlockSpec((lead, blk, 1), lambda i: (0, i, 0))

    def full_spec(a):
        return pl.BlockSpec(a.shape, lambda i: tuple(0 for _ in a.shape))

    in_specs = [
        bspec(U, 2 * D), bspec(F, D), bspec(S * F, 2 * D), bspec(1, D),
        mspec(U), mspec(F), mspec(S * F),
    ] + [full_spec(wa) for wa in weights]

    out = pl.pallas_call(
        functools.partial(_tc_body, U, F, S),
        grid=(grid,),
        in_specs=in_specs,
        out_specs=pl.BlockSpec((blk, D), lambda i: (i, 0)),
        out_shape=jax.ShapeDtypeStruct((B, D), jnp.float32),
    )(x1, xq, x3, g_iid, m1, mu, ms, *weights)
    return out
